# baseline (speedup 1.0000x reference)
import functools

import jax
import jax.numpy as jnp
from jax import lax
from jax.experimental import pallas as pl
from jax.experimental.pallas import tpu as pltpu
from jax.experimental.pallas import tpu_sc as plsc

N = 10000
E = 160000
H = 32
NC = 2
NS = 16
NW = NC * NS
C = 40
CH_PER_W = E // (NW * C)
N_PAD = 10240
RPT = N_PAD // NS
ZR = 128

_mesh = plsc.VectorSubcoreMesh(
    core_axis_name="c", subcore_axis_name="s", num_cores=NC, num_subcores=NS
)


@functools.partial(
    pl.kernel,
    out_type=jax.ShapeDtypeStruct((NC * N_PAD, H), jnp.float32),
    mesh=_mesh,
    scratch_types=[
        pltpu.VMEM((C, 8, 128), jnp.float32),
        pltpu.VMEM((C, 8, 128), jnp.float32),
        pltpu.VMEM((C, 128), jnp.float32),
        pltpu.VMEM((C, 128), jnp.float32),
        pltpu.VMEM((C, H), jnp.float32),
        pltpu.VMEM((CH_PER_W, 128), jnp.int32),
        pltpu.VMEM((C,), jnp.int32),
        pltpu.VMEM((C,), jnp.int32),
        pltpu.VMEM_SHARED((N_PAD, H), jnp.float32),
        pltpu.SemaphoreType.DMA,
        pltpu.SemaphoreType.DMA,
        pltpu.SemaphoreType.DMA,
        pltpu.SemaphoreType.DMA,
        pltpu.SemaphoreType.DMA,
        pltpu.SemaphoreType.DMA,
    ],
    compiler_params=pltpu.CompilerParams(use_tc_tiling_on_sc=False),
)
def _sc_edge_kernel(feat_hbm, src_hbm, dst_hbm, w_hbm, out_hbm,
                    wbuf0, wbuf1, xr0, xr1, msg, sidx, didx0, didx1,
                    accum, semw0, semw1, semg0, semg1, semd0, semd1):
    cid = lax.axis_index("c")
    sid = lax.axis_index("s")
    wid = sid * NC + cid
    zero16 = jnp.zeros((16,), jnp.float32)
    wbuf = (wbuf0, wbuf1)
    xr = (xr0, xr1)
    didx = (didx0, didx1)
    semw = (semw0, semw1)
    semg = (semg0, semg1)
    semd = (semd0, semd1)

    pltpu.sync_copy(src_hbm.at[pl.ds(wid * CH_PER_W, CH_PER_W)], sidx)

    def start_chunk(k, b):
        base = (wid * CH_PER_W + k) * C
        pltpu.async_copy(w_hbm.at[pl.ds(base, C)], wbuf[b], semw[b])
        pltpu.async_copy(feat_hbm.at[sidx.at[k, pl.ds(0, C)]], xr[b], semg[b])
        pltpu.async_copy(dst_hbm.at[wid * CH_PER_W + k, pl.ds(0, C)],
                         didx[b], semd[b])

    def wait_chunk(k, b):
        base = (wid * CH_PER_W + k) * C
        pltpu.make_async_copy(w_hbm.at[pl.ds(base, C)], wbuf[b], semw[b]).wait()
        pltpu.make_async_copy(feat_hbm.at[sidx.at[k, pl.ds(0, C)]], xr[b],
                              semg[b]).wait()

    def wait_didx(k, b):
        pltpu.make_async_copy(dst_hbm.at[wid * CH_PER_W + k, pl.ds(0, C)],
                              didx[b], semd[b]).wait()

    start_chunk(0, 0)

    def zrow(r, carry):
        msg[r, pl.ds(0, 16)] = zero16
        msg[r, pl.ds(16, 16)] = zero16
        return carry

    lax.fori_loop(0, C, zrow, 0)
    for q in range(RPT // C):
        pltpu.sync_copy(msg, accum.at[pl.ds(sid * RPT + q * C, C)])
    plsc.subcore_barrier()

    def compute_chunk(k, b):
        wb = wbuf[b]
        xb = xr[b]

        def edge_body(e, ecarry):
            a0 = zero16
            a1 = zero16
            a2 = zero16
            a3 = zero16
            x0 = xb[e, pl.ds(0, 16)]
            x1 = xb[e, pl.ds(16, 16)]
            for i in range(H):
                x = x0[i] if i < 16 else x1[i - 16]
                wlo = wb[e, i // 4, pl.ds((i % 4) * H, 16)]
                whi = wb[e, i // 4, pl.ds((i % 4) * H + 16, 16)]
                if i % 2 == 0:
                    a0 = a0 + x * wlo
                    a1 = a1 + x * whi
                else:
                    a2 = a2 + x * wlo
                    a3 = a3 + x * whi
            msg[e, pl.ds(0, 16)] = a0 + a2
            msg[e, pl.ds(16, 16)] = a1 + a3
            return ecarry

        lax.fori_loop(0, C, edge_body, 0)
        wait_didx(k, b)
        pltpu.sync_copy(msg, accum.at[didx[b]], add=True)

    def pair_body(k2, carry):
        for b in range(2):
            k = 2 * k2 + b
            start_chunk(k + 1, 1 - b)
            wait_chunk(k, b)
            compute_chunk(k, b)
        return carry

    lax.fori_loop(0, (CH_PER_W - 1) // 2, pair_body, 0)
    wait_chunk(CH_PER_W - 1, 0)
    compute_chunk(CH_PER_W - 1, 0)

    plsc.subcore_barrier()

    for q in range(RPT // ZR):
        r0 = sid * RPT + q * ZR
        pltpu.sync_copy(accum.at[pl.ds(r0, ZR)],
                        out_hbm.at[pl.ds(cid * N_PAD + r0, ZR)])


def _combine_body(f_ref, p_ref, o_ref):
    o_ref[...] = (f_ref[...] + p_ref[0] + p_ref[1]) * 0.5


_combine = pl.pallas_call(
    _combine_body,
    out_shape=jax.ShapeDtypeStruct((N * H // 128, 128), jnp.float32),
)


def kernel(feature, edge_index, edge_weight, W1, b1, W2, b2):
    src = jnp.pad(edge_index[0].reshape(E // C, C), ((0, 0), (0, 128 - C)))
    dst = jnp.pad(edge_index[1].reshape(E // C, C), ((0, 0), (0, 128 - C)))
    w3 = edge_weight.reshape(E, 8, 128)
    feat_p = jnp.pad(feature, ((0, 0), (0, 128 - H)))
    partial = _sc_edge_kernel(feat_p, src, dst, w3)
    p3 = partial.reshape(NC, N_PAD, H)[:, :N, :].reshape(NC, N * H // 128, 128)
    f2 = feature.reshape(N * H // 128, 128)
    out = _combine(f2, p3)
    return out.reshape(N, H)


# speedup vs baseline: 1.0243x; 1.0243x over previous
import functools

import jax
import jax.numpy as jnp
from jax import lax
from jax.experimental import pallas as pl
from jax.experimental.pallas import tpu as pltpu
from jax.experimental.pallas import tpu_sc as plsc

N = 10000
E = 160000
H = 32
NC = 2
NS = 16
NW = NC * NS
C = 40
CH_PER_W = E // (NW * C)
N_PAD = 10240
RPT = N_PAD // NS
ZR = 128

_mesh = plsc.VectorSubcoreMesh(
    core_axis_name="c", subcore_axis_name="s", num_cores=NC, num_subcores=NS
)


@functools.partial(
    pl.kernel,
    out_type=jax.ShapeDtypeStruct((NC * N_PAD * H // 128, 128), jnp.float32),
    mesh=_mesh,
    scratch_types=[
        pltpu.VMEM((C, 8, 128), jnp.float32),
        pltpu.VMEM((C, 8, 128), jnp.float32),
        pltpu.VMEM((C, 128), jnp.float32),
        pltpu.VMEM((C, 128), jnp.float32),
        pltpu.VMEM((C, H), jnp.float32),
        pltpu.VMEM((CH_PER_W, C), jnp.int32),
        pltpu.VMEM((C,), jnp.int32),
        pltpu.VMEM((C,), jnp.int32),
        pltpu.VMEM((64, H), jnp.float32),
        pltpu.VMEM((16, 128), jnp.float32),
        pltpu.VMEM_SHARED((N_PAD, H), jnp.float32),
        pltpu.SemaphoreType.DMA,
        pltpu.SemaphoreType.DMA,
        pltpu.SemaphoreType.DMA,
        pltpu.SemaphoreType.DMA,
        pltpu.SemaphoreType.DMA,
        pltpu.SemaphoreType.DMA,
    ],
    compiler_params=pltpu.CompilerParams(use_tc_tiling_on_sc=False),
)
def _sc_edge_kernel(feat_hbm, src_hbm, dst_hbm, w_hbm, out_hbm,
                    wbuf0, wbuf1, xr0, xr1, msg, sidx, didx0, didx1,
                    b_in, b_out, accum,
                    semw0, semw1, semg0, semg1, semd0, semd1):
    cid = lax.axis_index("c")
    sid = lax.axis_index("s")
    wid = sid * NC + cid
    zero16 = jnp.zeros((16,), jnp.float32)
    wbuf = (wbuf0, wbuf1)
    xr = (xr0, xr1)
    didx = (didx0, didx1)
    semw = (semw0, semw1)
    semg = (semg0, semg1)
    semd = (semd0, semd1)

    pltpu.sync_copy(src_hbm.at[pl.ds(wid * CH_PER_W, CH_PER_W), pl.ds(0, C)],
                    sidx)

    def start_chunk(k, b):
        base = (wid * CH_PER_W + k) * C
        pltpu.async_copy(w_hbm.at[pl.ds(base, C)], wbuf[b], semw[b])
        pltpu.async_copy(feat_hbm.at[sidx.at[k]], xr[b], semg[b])
        pltpu.async_copy(dst_hbm.at[wid * CH_PER_W + k, pl.ds(0, C)],
                         didx[b], semd[b])

    def wait_chunk(k, b):
        base = (wid * CH_PER_W + k) * C
        pltpu.make_async_copy(w_hbm.at[pl.ds(base, C)], wbuf[b], semw[b]).wait()
        pltpu.make_async_copy(feat_hbm.at[sidx.at[k]], xr[b], semg[b]).wait()

    def wait_didx(k, b):
        pltpu.make_async_copy(dst_hbm.at[wid * CH_PER_W + k, pl.ds(0, C)],
                              didx[b], semd[b]).wait()

    start_chunk(0, 0)

    def zrow(r, carry):
        msg[r, pl.ds(0, 16)] = zero16
        msg[r, pl.ds(16, 16)] = zero16
        return carry

    lax.fori_loop(0, C, zrow, 0)
    for q in range(RPT // C):
        pltpu.sync_copy(msg, accum.at[pl.ds(sid * RPT + q * C, C)])
    plsc.subcore_barrier()

    def compute_chunk(k, b):
        wb = wbuf[b]
        xb = xr[b]

        def edge_body(e, ecarry):
            a0 = zero16
            a1 = zero16
            a2 = zero16
            a3 = zero16
            x0 = xb[e, pl.ds(0, 16)]
            x1 = xb[e, pl.ds(16, 16)]
            for i in range(H):
                x = x0[i] if i < 16 else x1[i - 16]
                wlo = wb[e, i // 4, pl.ds((i % 4) * H, 16)]
                whi = wb[e, i // 4, pl.ds((i % 4) * H + 16, 16)]
                if i % 2 == 0:
                    a0 = a0 + x * wlo
                    a1 = a1 + x * whi
                else:
                    a2 = a2 + x * wlo
                    a3 = a3 + x * whi
            msg[e, pl.ds(0, 16)] = a0 + a2
            msg[e, pl.ds(16, 16)] = a1 + a3
            return ecarry

        lax.fori_loop(0, C, edge_body, 0)
        wait_didx(k, b)
        pltpu.sync_copy(msg, accum.at[didx[b]], add=True)

    def pair_body(k2, carry):
        for b in range(2):
            k = 2 * k2 + b
            start_chunk(k + 1, 1 - b)
            wait_chunk(k, b)
            compute_chunk(k, b)
        return carry

    lax.fori_loop(0, (CH_PER_W - 1) // 2, pair_body, 0)
    wait_chunk(CH_PER_W - 1, 0)
    compute_chunk(CH_PER_W - 1, 0)

    plsc.subcore_barrier()

    def repack(rr, carry):
        lo = b_in[rr, pl.ds(0, 16)]
        hi = b_in[rr, pl.ds(16, 16)]
        b_out[rr // 4, pl.ds((rr % 4) * H, 16)] = lo
        b_out[rr // 4, pl.ds((rr % 4) * H + 16, 16)] = hi
        return carry

    for q in range(RPT // 64):
        r0 = sid * RPT + q * 64
        pltpu.sync_copy(accum.at[pl.ds(r0, 64)], b_in)
        lax.fori_loop(0, 64, repack, 0)
        orow = (cid * N_PAD + r0) * H // 128
        pltpu.sync_copy(b_out, out_hbm.at[pl.ds(orow, 16)])


def _combine_body(f_ref, p_ref, o_ref):
    o_ref[...] = (f_ref[...] + p_ref[0] + p_ref[1]) * 0.5


_combine = pl.pallas_call(
    _combine_body,
    out_shape=jax.ShapeDtypeStruct((N * H // 128, 128), jnp.float32),
)


def kernel(feature, edge_index, edge_weight, W1, b1, W2, b2):
    src = jnp.pad(edge_index[0].reshape(E // C, C), ((0, 0), (0, 128 - C)))
    dst = jnp.pad(edge_index[1].reshape(E // C, C), ((0, 0), (0, 128 - C)))
    w3 = edge_weight.reshape(E, 8, 128)
    feat_p = jnp.pad(feature, ((0, 0), (0, 128 - H)))
    partial = _sc_edge_kernel(feat_p, src, dst, w3)
    p3 = partial.reshape(NC, N_PAD * H // 128, 128)[:, :N * H // 128, :]
    f2 = feature.reshape(N * H // 128, 128)
    out = _combine(f2, p3)
    return out.reshape(N, H)
